# baseline (speedup 1.0000x reference)
import functools

import jax
import jax.numpy as jnp
from jax import lax
from jax.experimental import pallas as pl
from jax.experimental.pallas import tpu as pltpu
from jax.experimental.pallas import tpu_sc as plsc

M_SUB = 16
K_CODES = 512
D_SUB = 64

NC = 2
NS = 16
NW = NC * NS


def _matmul_kernel(x_ref, w_ref, out_ref):
    xb = x_ref[...].astype(jnp.bfloat16)
    wb = w_ref[...].astype(jnp.bfloat16)
    out_ref[...] = jax.lax.dot_general(
        xb, wb, (((1,), (1,)), ((), ())),
        preferred_element_type=jnp.float32)


def _pq_idx_kernel(x_ref, cbt_ref, even_ref, odd_ref):
    B = x_ref.shape[0]
    k_iota = jax.lax.broadcasted_iota(jnp.int32, (B, K_CODES), 1)
    for m in range(M_SUB):
        xs = x_ref[:, m * D_SUB:(m + 1) * D_SUB]
        cbt = cbt_ref[m]
        c2 = jnp.sum(cbt * cbt, axis=0)
        xc = jnp.dot(xs, cbt,
                     preferred_element_type=jnp.float32)
        d = c2[None, :] - 2.0 * xc
        min_d = jnp.min(d, axis=1, keepdims=True)
        idx = jnp.min(jnp.where(d == min_d, k_iota, K_CODES),
                      axis=1, keepdims=True)
        j = m // 2
        if m % 2 == 0:
            even_ref[:, j:j + 1] = idx + m * K_CODES
        else:
            odd_ref[:, j:j + 1] = idx + m * K_CODES


def _make_sc_gather(n_rows128):
    rpw = n_rows128 // NW
    crows = 128
    ch = rpw // crows
    mesh = plsc.VectorSubcoreMesh(core_axis_name="c", subcore_axis_name="s")

    @functools.partial(
        pl.kernel, mesh=mesh,
        out_type=jax.ShapeDtypeStruct((NW, ch, crows, 128), jnp.float32),
        scratch_types=[
            pltpu.VMEM((ch, crows), jnp.int32),
            pltpu.VMEM((ch, crows), jnp.int32),
            pltpu.VMEM((crows, 128), jnp.float32),
            pltpu.VMEM((crows, 128), jnp.float32),
            pltpu.SemaphoreType.DMA,
            pltpu.SemaphoreType.DMA,
        ],
    )
    def sc_gather(table_hbm, ide_hbm, ido_hbm, out_hbm,
                  ide_v, ido_v, bufa, bufb, sema, semb):
        wid = lax.axis_index("s") * NC + lax.axis_index("c")
        pltpu.sync_copy(ide_hbm.at[wid], ide_v)
        pltpu.sync_copy(ido_hbm.at[wid], ido_v)
        for c in range(ch):
            cpa = pltpu.async_copy(table_hbm.at[ide_v.at[c]], bufa, sema)
            cpb = pltpu.async_copy(table_hbm.at[ido_v.at[c]], bufb, semb)
            cpa.wait()
            cpb.wait()

            def row_body(r, carry):
                for jj in range(D_SUB // 16):
                    sl = pl.ds(D_SUB + jj * 16, 16)
                    bufa[r, sl] = bufb[r, sl]
                return carry

            lax.fori_loop(0, crows, row_body, 0)
            pltpu.sync_copy(bufa, out_hbm.at[wid, c])

    return sc_gather, ch, crows


def kernel(x, weight, codebooks):
    N, D = x.shape
    OUT = weight.shape[0]

    BQ = 256
    cbt = jnp.swapaxes(codebooks, 1, 2)
    idx_even, idx_odd = pl.pallas_call(
        _pq_idx_kernel,
        grid=(N // BQ,),
        in_specs=[
            pl.BlockSpec((BQ, D), lambda i: (i, 0)),
            pl.BlockSpec((M_SUB, D_SUB, K_CODES), lambda i: (0, 0, 0)),
        ],
        out_specs=[
            pl.BlockSpec((BQ, M_SUB // 2), lambda i: (i, 0)),
            pl.BlockSpec((BQ, M_SUB // 2), lambda i: (i, 0)),
        ],
        out_shape=[
            jax.ShapeDtypeStruct((N, M_SUB // 2), jnp.int32),
            jax.ShapeDtypeStruct((N, M_SUB // 2), jnp.int32),
        ],
    )(x, cbt)

    cb_flat = codebooks.reshape(M_SUB * K_CODES, D_SUB)
    table = jnp.concatenate([cb_flat, cb_flat], axis=1)

    n_rows128 = N * M_SUB // 2
    sc_gather, ch, crows = _make_sc_gather(n_rows128)
    ide = idx_even.reshape(NW, ch, crows)
    ido = idx_odd.reshape(NW, ch, crows)
    xq = sc_gather(table, ide, ido).reshape(N, D)

    BM = 512
    out = pl.pallas_call(
        _matmul_kernel,
        grid=(N // BM,),
        in_specs=[
            pl.BlockSpec((BM, D), lambda i: (i, 0)),
            pl.BlockSpec((OUT, D), lambda i: (0, 0)),
        ],
        out_specs=pl.BlockSpec((BM, OUT), lambda i: (i, 0)),
        out_shape=jax.ShapeDtypeStruct((N, OUT), jnp.float32),
    )(x, weight)

    return (out, xq)


# speedup vs baseline: 1.2486x; 1.2486x over previous
import functools

import jax
import jax.numpy as jnp
from jax import lax
from jax.experimental import pallas as pl
from jax.experimental.pallas import tpu as pltpu
from jax.experimental.pallas import tpu_sc as plsc

M_SUB = 16
K_CODES = 512
D_SUB = 64

NC = 2
NS = 16
NW = NC * NS


def _matmul_kernel(x_ref, w_ref, out_ref):
    xb = x_ref[...].astype(jnp.bfloat16)
    wb = w_ref[...].astype(jnp.bfloat16)
    out_ref[...] = jax.lax.dot_general(
        xb, wb, (((1,), (1,)), ((), ())),
        preferred_element_type=jnp.float32)


def _pq_idx_kernel(x_ref, cbt_ref, even_ref, odd_ref):
    B = x_ref.shape[0]
    kf_iota = jax.lax.broadcasted_iota(
        jnp.int32, (B, K_CODES), 1).astype(jnp.float32)
    for m in range(M_SUB):
        xs = x_ref[:, m * D_SUB:(m + 1) * D_SUB]
        cbt = cbt_ref[m]
        c2 = jnp.sum(cbt * cbt, axis=0)
        xc = jnp.dot(xs, cbt,
                     preferred_element_type=jnp.float32)
        d = c2[None, :] - 2.0 * xc
        min_d = jnp.min(d, axis=1, keepdims=True)
        idxf = jnp.min(jnp.where(d == min_d, kf_iota, float(K_CODES)),
                       axis=1, keepdims=True)
        idx = idxf.astype(jnp.int32)
        j = m // 2
        if m % 2 == 0:
            even_ref[:, j:j + 1] = idx + m * K_CODES
        else:
            odd_ref[:, j:j + 1] = idx + m * K_CODES


def _make_sc_gather(n_rows128):
    rpw = n_rows128 // NW
    crows = 128
    ch = rpw // crows
    mesh = plsc.VectorSubcoreMesh(core_axis_name="c", subcore_axis_name="s")

    @functools.partial(
        pl.kernel, mesh=mesh,
        out_type=jax.ShapeDtypeStruct((NW, ch, crows, 128), jnp.float32),
        scratch_types=[
            pltpu.VMEM((ch, crows), jnp.int32),
            pltpu.VMEM((ch, crows), jnp.int32),
            pltpu.VMEM((2, crows, 128), jnp.float32),
            pltpu.VMEM((2, crows, 128), jnp.float32),
            pltpu.SemaphoreType.DMA,
            pltpu.SemaphoreType.DMA,
            pltpu.SemaphoreType.DMA,
            pltpu.SemaphoreType.DMA,
        ],
    )
    def sc_gather(table_hbm, ide_hbm, ido_hbm, out_hbm,
                  ide_v, ido_v, bufa, bufb, sema0, sema1, semb0, semb1):
        wid = lax.axis_index("s") * NC + lax.axis_index("c")
        sems = ((sema0, semb0), (sema1, semb1))
        pltpu.sync_copy(ide_hbm.at[wid], ide_v)
        pltpu.sync_copy(ido_hbm.at[wid], ido_v)

        def start(c):
            s = c % 2
            cpa = pltpu.async_copy(
                table_hbm.at[ide_v.at[c]], bufa.at[s], sems[s][0])
            cpb = pltpu.async_copy(
                table_hbm.at[ido_v.at[c]], bufb.at[s], sems[s][1])
            return cpa, cpb

        pend = start(0)
        for c in range(ch):
            s = c % 2
            cpa, cpb = pend
            cpa.wait()
            cpb.wait()
            if c + 1 < ch:
                pend = start(c + 1)

            def row_body(r, carry):
                for jj in range(D_SUB // 16):
                    sl = pl.ds(D_SUB + jj * 16, 16)
                    bufa[s, r, sl] = bufb[s, r, sl]
                return carry

            lax.fori_loop(0, crows, row_body, 0)
            pltpu.sync_copy(bufa.at[s], out_hbm.at[wid, c])

    return sc_gather, ch, crows


def kernel(x, weight, codebooks):
    N, D = x.shape
    OUT = weight.shape[0]

    BQ = 256
    cbt = jnp.swapaxes(codebooks, 1, 2)
    idx_even, idx_odd = pl.pallas_call(
        _pq_idx_kernel,
        grid=(N // BQ,),
        in_specs=[
            pl.BlockSpec((BQ, D), lambda i: (i, 0)),
            pl.BlockSpec((M_SUB, D_SUB, K_CODES), lambda i: (0, 0, 0)),
        ],
        out_specs=[
            pl.BlockSpec((BQ, M_SUB // 2), lambda i: (i, 0)),
            pl.BlockSpec((BQ, M_SUB // 2), lambda i: (i, 0)),
        ],
        out_shape=[
            jax.ShapeDtypeStruct((N, M_SUB // 2), jnp.int32),
            jax.ShapeDtypeStruct((N, M_SUB // 2), jnp.int32),
        ],
    )(x, cbt)

    cb_flat = codebooks.reshape(M_SUB * K_CODES, D_SUB)
    table = jnp.concatenate([cb_flat, cb_flat], axis=1)

    n_rows128 = N * M_SUB // 2
    sc_gather, ch, crows = _make_sc_gather(n_rows128)
    ide = idx_even.reshape(NW, ch, crows)
    ido = idx_odd.reshape(NW, ch, crows)
    xq = sc_gather(table, ide, ido).reshape(N, D)

    BM = 512
    out = pl.pallas_call(
        _matmul_kernel,
        grid=(N // BM,),
        in_specs=[
            pl.BlockSpec((BM, D), lambda i: (i, 0)),
            pl.BlockSpec((OUT, D), lambda i: (0, 0)),
        ],
        out_specs=pl.BlockSpec((BM, OUT), lambda i: (i, 0)),
        out_shape=jax.ShapeDtypeStruct((N, OUT), jnp.float32),
    )(x, weight)

    return (out, xq)
